# Initial kernel scaffold; baseline (speedup 1.0000x reference)
import jax
import jax.numpy as jnp
from jax.experimental import pallas as pl


def kernel(user_emb, item_emb, adj_values, adj_indices):
    raise NotImplementedError("write your pallas kernel here")


# speedup vs baseline: 3.4005x; 3.4005x over previous
import jax
import jax.numpy as jnp
from jax import lax
from jax.experimental import pallas as pl
from jax.experimental.pallas import tpu as pltpu
from jax.experimental.pallas import tpu_sc as plsc

USER_NUM = 25000
ITEM_NUM = 25000
N_NODES = USER_NUM + ITEM_NUM
N_EDGES = 800000
EMB = 64
QC = 16
NQ = 4
N_LAYERS = 3

NC = 2
NS = 16
CH = 128
JJ = 8
GROUP = JJ * CH
G = 49
EPT = G * GROUP
EP = NS * EPT
PT = N_NODES // NS


def _spmm_body(table, colh, lidxh, valh, zrows, out,
               colv, lidxv, valv, gbuf, acc, sem):
    c = lax.axis_index("c")
    s = lax.axis_index("s")

    row_base = s * (EPT // CH)
    flat_base = s * EPT

    for p in range(2):
        q = c * 2 + p
        bias = q * N_NODES

        pltpu.sync_copy(zrows, acc.at[pl.ds(s * PT, PT)])
        plsc.subcore_barrier()

        def group_loop(g, _):
            roff = row_base + g * JJ
            foff = flat_base + g * GROUP
            pltpu.sync_copy(colh.at[pl.ds(roff, JJ)], colv)
            pltpu.sync_copy(lidxh.at[pl.ds(roff, JJ)], lidxv)
            pltpu.sync_copy(valh.at[pl.ds(foff, GROUP)], valv)

            def bias_loop(bq, _):
                j2 = bq // JJ
                k2 = bq % JJ
                cv = colv[j2, pl.ds(k2 * 16, 16)]
                colv[j2, pl.ds(k2 * 16, 16)] = cv + bias
                return 0

            lax.fori_loop(0, JJ * JJ, bias_loop, 0)

            def j_loop(j, _):
                dst = gbuf.at[pl.ds(j * CH, CH)]
                pltpu.async_copy(table.at[colv.at[j]], dst, sem).wait()

                def e_loop(e16, _):
                    vv = valv[pl.ds(j * CH + e16 * 16, 16)]
                    for l in range(16):
                        r = j * CH + e16 * 16 + l
                        gbuf[r, :] = gbuf[r, :] * vv[l]
                    return 0

                lax.fori_loop(0, CH // 16, e_loop, 0)
                pltpu.sync_copy(gbuf.at[pl.ds(j * CH, CH)],
                                acc.at[lidxv.at[j]], add=True)
                return 0

            lax.fori_loop(0, JJ, j_loop, 0)
            return 0

        lax.fori_loop(0, G, group_loop, 0)
        plsc.subcore_barrier()

        pltpu.sync_copy(acc.at[pl.ds(s * PT, PT)],
                        out.at[pl.ds(q * N_NODES + s * PT, PT)])


_spmm = pl.kernel(
    _spmm_body,
    out_type=jax.ShapeDtypeStruct((NQ * N_NODES, QC), jnp.float32),
    mesh=plsc.VectorSubcoreMesh(core_axis_name="c", subcore_axis_name="s"),
    scratch_types=[
        pltpu.VMEM((JJ, CH), jnp.int32),
        pltpu.VMEM((JJ, CH), jnp.int32),
        pltpu.VMEM((GROUP,), jnp.float32),
        pltpu.VMEM((GROUP, QC), jnp.float32),
        pltpu.VMEM_SHARED((N_NODES, QC), jnp.float32),
        pltpu.SemaphoreType.DMA,
    ],
    compiler_params=pltpu.CompilerParams(use_tc_tiling_on_sc=False),
)


def _mean3_body(a, b, c, o):
    o[...] = (a[...] + b[...] + c[...]) * (1.0 / 3.0)


def _mean3(a, b, c):
    rows = NQ * N_NODES * QC // 128
    a = a.reshape(rows, 128)
    b = b.reshape(rows, 128)
    c = c.reshape(rows, 128)
    blk = 1000
    spec = pl.BlockSpec((blk, 128), lambda i: (i, 0))
    out = pl.pallas_call(
        _mean3_body,
        grid=(rows // blk,),
        in_specs=[spec, spec, spec],
        out_specs=spec,
        out_shape=jax.ShapeDtypeStruct((rows, 128), jnp.float32),
    )(a, b, c)
    return out.reshape(NQ * N_NODES, QC)


def kernel(user_emb, item_emb, adj_values, adj_indices):
    row = adj_indices[0].astype(jnp.int32)
    col = adj_indices[1].astype(jnp.int32)
    val = adj_values.astype(jnp.float32)

    pad = EP - N_EDGES
    colp = jnp.concatenate([col, jnp.zeros((pad,), jnp.int32)])
    lidxp = jnp.concatenate([row, jnp.zeros((pad,), jnp.int32)])
    valp = jnp.concatenate([val, jnp.zeros((pad,), jnp.float32)])
    colh = colp.reshape(EP // CH, CH)
    lidxh = lidxp.reshape(EP // CH, CH)

    ego = jnp.concatenate([user_emb, item_emb], axis=0)
    table = jnp.concatenate(
        [ego[:, 0:16], ego[:, 16:32], ego[:, 32:48], ego[:, 48:64]], axis=0)

    zrows = jnp.zeros((PT, QC), jnp.float32)

    layers = []
    for _ in range(N_LAYERS):
        table = _spmm(table, colh, lidxh, valp, zrows)
        layers.append(table)

    m = _mean3(*layers)

    user = jnp.concatenate(
        [m[i * N_NODES:i * N_NODES + USER_NUM] for i in range(NQ)], axis=1)
    item = jnp.concatenate(
        [m[i * N_NODES + USER_NUM:(i + 1) * N_NODES] for i in range(NQ)],
        axis=1)
    return (user, item)


# speedup vs baseline: 7.2677x; 2.1373x over previous
import jax
import jax.numpy as jnp
from jax import lax
from jax.experimental import pallas as pl
from jax.experimental.pallas import tpu as pltpu
from jax.experimental.pallas import tpu_sc as plsc

USER_NUM = 25000
ITEM_NUM = 25000
N_NODES = USER_NUM + ITEM_NUM
N_EDGES = 800000
EMB = 64
QC = 16
NQ = 4
N_LAYERS = 3

NC = 2
NS = 16
CH = 128
JJ = 8
GROUP = JJ * CH
G = 49
EPT = G * GROUP
EP = NS * EPT
PT = N_NODES // NS


def _spmm_body(table, colh, lidxh, valh, zrows, out,
               colv, lidxv, valv, gbuf, acc, gsem, ssem):
    c = lax.axis_index("c")
    s = lax.axis_index("s")

    row_base = s * (EPT // CH)
    flat_base = s * EPT

    def drain_scatter(j):
        pltpu.make_async_copy(
            gbuf.at[pl.ds(j * CH, CH)],
            acc.at[pl.ds(j * CH, CH)], ssem.at[j]).wait()

    for p in range(2):
        q = c * 2 + p

        pltpu.sync_copy(zrows, acc.at[pl.ds(s * PT, PT)])
        plsc.subcore_barrier()

        def group_loop(g, _):
            @pl.when(g > 0)
            def _():
                for j in range(JJ):
                    drain_scatter(j)

            roff = row_base + g * JJ
            foff = flat_base + g * GROUP
            pltpu.sync_copy(colh.at[q, pl.ds(roff, JJ)], colv)
            pltpu.sync_copy(lidxh.at[pl.ds(roff, JJ)], lidxv)
            pltpu.sync_copy(valh.at[pl.ds(foff, GROUP)], valv)

            gds = [pltpu.async_copy(table.at[colv.at[j]],
                                    gbuf.at[pl.ds(j * CH, CH)], gsem.at[j])
                   for j in range(JJ)]

            for j in range(JJ):
                gds[j].wait()

                def e_loop(e16, _):
                    vv = valv[pl.ds(j * CH + e16 * 16, 16)]
                    for l in range(16):
                        r = j * CH + e16 * 16 + l
                        gbuf[r, :] = gbuf[r, :] * vv[l]
                    return 0

                lax.fori_loop(0, CH // 16, e_loop, 0)
                pltpu.async_copy(gbuf.at[pl.ds(j * CH, CH)],
                                 acc.at[lidxv.at[j]], ssem.at[j], add=True)
            return 0

        lax.fori_loop(0, G, group_loop, 0)
        for j in range(JJ):
            drain_scatter(j)
        plsc.subcore_barrier()

        pltpu.sync_copy(acc.at[pl.ds(s * PT, PT)],
                        out.at[pl.ds(q * N_NODES + s * PT, PT)])


_spmm = pl.kernel(
    _spmm_body,
    out_type=jax.ShapeDtypeStruct((NQ * N_NODES, QC), jnp.float32),
    mesh=plsc.VectorSubcoreMesh(core_axis_name="c", subcore_axis_name="s"),
    scratch_types=[
        pltpu.VMEM((JJ, CH), jnp.int32),
        pltpu.VMEM((JJ, CH), jnp.int32),
        pltpu.VMEM((GROUP,), jnp.float32),
        pltpu.VMEM((GROUP, QC), jnp.float32),
        pltpu.VMEM_SHARED((N_NODES, QC), jnp.float32),
        pltpu.SemaphoreType.DMA((JJ,)),
        pltpu.SemaphoreType.DMA((JJ,)),
    ],
    compiler_params=pltpu.CompilerParams(use_tc_tiling_on_sc=False),
)


def _mean3_body(a, b, c, o):
    o[...] = (a[...] + b[...] + c[...]) * (1.0 / 3.0)


def _mean3(a, b, c):
    rows = NQ * N_NODES * QC // 128
    a = a.reshape(rows, 128)
    b = b.reshape(rows, 128)
    c = c.reshape(rows, 128)
    blk = 1000
    spec = pl.BlockSpec((blk, 128), lambda i: (i, 0))
    out = pl.pallas_call(
        _mean3_body,
        grid=(rows // blk,),
        in_specs=[spec, spec, spec],
        out_specs=spec,
        out_shape=jax.ShapeDtypeStruct((rows, 128), jnp.float32),
    )(a, b, c)
    return out.reshape(NQ * N_NODES, QC)


def kernel(user_emb, item_emb, adj_values, adj_indices):
    row = adj_indices[0].astype(jnp.int32)
    col = adj_indices[1].astype(jnp.int32)
    val = adj_values.astype(jnp.float32)

    pad = EP - N_EDGES
    colp = jnp.concatenate([col, jnp.zeros((pad,), jnp.int32)])
    lidxp = jnp.concatenate([row, jnp.zeros((pad,), jnp.int32)])
    valp = jnp.concatenate([val, jnp.zeros((pad,), jnp.float32)])
    offs = (jnp.arange(NQ, dtype=jnp.int32) * N_NODES)[:, None]
    colh = (colp[None, :] + offs).reshape(NQ, EP // CH, CH)
    lidxh = lidxp.reshape(EP // CH, CH)

    ego = jnp.concatenate([user_emb, item_emb], axis=0)
    table = jnp.concatenate(
        [ego[:, 0:16], ego[:, 16:32], ego[:, 32:48], ego[:, 48:64]], axis=0)

    zrows = jnp.zeros((PT, QC), jnp.float32)

    layers = []
    for _ in range(N_LAYERS):
        table = _spmm(table, colh, lidxh, valp, zrows)
        layers.append(table)

    m = _mean3(*layers)

    user = jnp.concatenate(
        [m[i * N_NODES:i * N_NODES + USER_NUM] for i in range(NQ)], axis=1)
    item = jnp.concatenate(
        [m[i * N_NODES + USER_NUM:(i + 1) * N_NODES] for i in range(NQ)],
        axis=1)
    return (user, item)
